# baseline (speedup 1.0000x reference)
import functools

import jax
import jax.numpy as jnp
from jax import lax
from jax.experimental import pallas as pl
from jax.experimental.pallas import tpu as pltpu
from jax.experimental.pallas import tpu_sc as plsc

_VOCAB = 1000000
_EMBED_DIM = 64
_BATCH = 16384

_INFO = plsc.get_sparse_core_info()
_NC, _NS, _L = _INFO.num_cores, _INFO.num_subcores, _INFO.num_lanes
_NW = _NC * _NS
_BPW = _BATCH // _NW
_CHUNK = 128
_NCHUNK = _BPW // _CHUNK


def _sc_body(tgt_hbm, ctx_hbm, table_hbm, out_hbm,
             idx_t, idx_c, rows_t, rows_c, out_v, sem):
    wid = lax.axis_index("s") * _NC + lax.axis_index("c")
    row0 = wid * _NCHUNK

    pltpu.sync_copy(tgt_hbm.at[pl.ds(row0, _NCHUNK)], idx_t)
    pltpu.sync_copy(ctx_hbm.at[pl.ds(row0, _NCHUNK)], idx_c)

    copies = []
    for j in range(_NCHUNK):
        copies.append(pltpu.async_copy(
            table_hbm.at[idx_t.at[j]],
            rows_t.at[pl.ds(j * _CHUNK, _CHUNK)], sem))
        copies.append(pltpu.async_copy(
            table_hbm.at[idx_c.at[j]],
            rows_c.at[pl.ds(j * _CHUNK, _CHUNK)], sem))
    for c in copies:
        c.wait()

    lane = lax.iota(jnp.int32, _L)

    def body(g, carry):
        row_ids = g * _L + lane
        acc = jnp.zeros((_L,), jnp.float32)
        for d in range(_EMBED_DIM):
            dim_ids = jnp.full((_L,), d, jnp.int32)
            t = plsc.load_gather(rows_t, [row_ids, dim_ids])
            c = plsc.load_gather(rows_c, [row_ids, dim_ids])
            acc = acc + t * c
        e = jnp.exp(-jnp.abs(acc))
        r = 1.0 / (1.0 + e)
        sig = jnp.where(acc >= 0, r, e * r)
        out_v[pl.ds(g * _L, _L)] = sig
        return carry

    lax.fori_loop(0, _BPW // _L, body, 0)

    pltpu.sync_copy(out_v, out_hbm.at[pl.ds(wid * _BPW, _BPW)])


@jax.jit
def _run(target_i, context_j, shared_embedding):
    mesh = plsc.VectorSubcoreMesh(core_axis_name="c", subcore_axis_name="s")
    tgt2d = target_i.reshape(_NW * _NCHUNK, _CHUNK)
    ctx2d = context_j.reshape(_NW * _NCHUNK, _CHUNK)
    kern = functools.partial(
        pl.kernel,
        out_type=jax.ShapeDtypeStruct((_BATCH,), jnp.float32),
        mesh=mesh,
        scratch_types=[
            pltpu.VMEM((_NCHUNK, _CHUNK), jnp.int32),
            pltpu.VMEM((_NCHUNK, _CHUNK), jnp.int32),
            pltpu.VMEM((_BPW, _EMBED_DIM), jnp.float32),
            pltpu.VMEM((_BPW, _EMBED_DIM), jnp.float32),
            pltpu.VMEM((_BPW,), jnp.float32),
            pltpu.SemaphoreType.DMA,
        ],
        compiler_params=pltpu.CompilerParams(
            needs_layout_passes=False, use_tc_tiling_on_sc=False),
    )(_sc_body)
    return kern(tgt2d, ctx2d, shared_embedding)


def kernel(target_i, context_j, shared_embedding):
    return _run(target_i.astype(jnp.int32), context_j.astype(jnp.int32),
                shared_embedding)
